# Initial kernel scaffold; baseline (speedup 1.0000x reference)
import jax
import jax.numpy as jnp
from jax.experimental import pallas as pl


def kernel(x, y_true, prototypes, counter):
    raise NotImplementedError("write your pallas kernel here")


# speedup vs baseline: 1.1795x; 1.1795x over previous
import functools

import jax
import jax.numpy as jnp
from jax.experimental import pallas as pl
from jax.experimental.pallas import tpu as pltpu

N_TOKENS = 8192
NUM_CLASSES = 1024
EMBED_DIM = 128

A_BLK = 512
B_BLK = 512
A_STEPS = N_TOKENS // A_BLK
B_STEPS = N_TOKENS // B_BLK

_PREC = jax.lax.Precision.HIGHEST


def _update_kernel(x_ref, y_ref, p_ref, c_ref, u_ref, sums_ref, cnt_ref):
    i = pl.program_id(0)

    @pl.when(i == 0)
    def _init():
        sums_ref[...] = jnp.zeros_like(sums_ref)
        cnt_ref[...] = jnp.zeros_like(cnt_ref)

    y_blk = y_ref[...]
    cls = jax.lax.broadcasted_iota(jnp.int32, (A_BLK, NUM_CLASSES), 1)
    oh = (y_blk == cls).astype(jnp.float32)
    sums_ref[...] += jax.lax.dot_general(
        oh, x_ref[...], (((0,), (0,)), ((), ())),
        precision=_PREC, preferred_element_type=jnp.float32)
    cnt_ref[...] += jax.lax.dot_general(
        oh, jnp.ones((A_BLK, EMBED_DIM), jnp.float32), (((0,), (0,)), ((), ())),
        precision=_PREC, preferred_element_type=jnp.float32)

    @pl.when(i == A_STEPS - 1)
    def _finish():
        cnt = cnt_ref[...]
        new = sums_ref[...] / jnp.maximum(cnt, 1.0)
        c = c_ref[...]
        u_ref[...] = jnp.where(cnt > 0.0,
                               (c * p_ref[...] + new) / (c + 1.0),
                               p_ref[...])


def _dist_kernel(x_ref, u_ref, o_ref):
    x = x_ref[...]
    u = u_ref[...]
    ones_row = jnp.ones((1, EMBED_DIM), jnp.float32)
    d = jax.lax.dot_general(x, u, (((1,), (1,)), ((), ())),
                            precision=_PREC, preferred_element_type=jnp.float32)
    xsq = jax.lax.dot_general(x * x, ones_row, (((1,), (1,)), ((), ())),
                              precision=_PREC, preferred_element_type=jnp.float32)
    usq = jax.lax.dot_general(ones_row, u * u, (((1,), (1,)), ((), ())),
                              precision=_PREC, preferred_element_type=jnp.float32)
    o_ref[...] = -jnp.maximum(xsq + usq - 2.0 * d, 0.0)


def kernel(x, y_true, prototypes, counter):
    y2 = y_true.reshape(N_TOKENS, 1)
    c2 = counter.reshape(NUM_CLASSES, 1)

    u = pl.pallas_call(
        _update_kernel,
        grid=(A_STEPS,),
        in_specs=[
            pl.BlockSpec((A_BLK, EMBED_DIM), lambda i: (i, 0)),
            pl.BlockSpec((A_BLK, 1), lambda i: (i, 0)),
            pl.BlockSpec((NUM_CLASSES, EMBED_DIM), lambda i: (0, 0)),
            pl.BlockSpec((NUM_CLASSES, 1), lambda i: (0, 0)),
        ],
        out_specs=pl.BlockSpec((NUM_CLASSES, EMBED_DIM), lambda i: (0, 0)),
        out_shape=jax.ShapeDtypeStruct((NUM_CLASSES, EMBED_DIM), jnp.float32),
        scratch_shapes=[
            pltpu.VMEM((NUM_CLASSES, EMBED_DIM), jnp.float32),
            pltpu.VMEM((NUM_CLASSES, EMBED_DIM), jnp.float32),
        ],
    )(x, y2, prototypes, c2)

    out = pl.pallas_call(
        _dist_kernel,
        grid=(B_STEPS,),
        in_specs=[
            pl.BlockSpec((B_BLK, EMBED_DIM), lambda i: (i, 0)),
            pl.BlockSpec((NUM_CLASSES, EMBED_DIM), lambda i: (0, 0)),
        ],
        out_specs=pl.BlockSpec((B_BLK, NUM_CLASSES), lambda i: (i, 0)),
        out_shape=jax.ShapeDtypeStruct((N_TOKENS, NUM_CLASSES), jnp.float32),
    )(x, u)
    return out


# speedup vs baseline: 2.8940x; 2.4537x over previous
import functools

import jax
import jax.numpy as jnp
from jax.experimental import pallas as pl
from jax.experimental.pallas import tpu as pltpu

N_TOKENS = 8192
NUM_CLASSES = 1024
EMBED_DIM = 128

A_BLK = 512
B_BLK = 512
A_STEPS = N_TOKENS // A_BLK
B_STEPS = N_TOKENS // B_BLK

_PREC = jax.lax.Precision.DEFAULT


def _update_kernel(x_ref, y_ref, p_ref, c_ref, u_ref, sums_ref, cnt_ref):
    i = pl.program_id(0)

    @pl.when(i == 0)
    def _init():
        sums_ref[...] = jnp.zeros_like(sums_ref)
        cnt_ref[...] = jnp.zeros_like(cnt_ref)

    y_blk = y_ref[...]
    cls = jax.lax.broadcasted_iota(jnp.int32, (A_BLK, NUM_CLASSES), 1)
    oh = (y_blk == cls).astype(jnp.float32)
    sums_ref[...] += jax.lax.dot_general(
        oh, x_ref[...], (((0,), (0,)), ((), ())),
        precision=_PREC, preferred_element_type=jnp.float32)
    cnt_ref[...] += jax.lax.dot_general(
        oh, jnp.ones((A_BLK, EMBED_DIM), jnp.float32), (((0,), (0,)), ((), ())),
        precision=_PREC, preferred_element_type=jnp.float32)

    @pl.when(i == A_STEPS - 1)
    def _finish():
        cnt = cnt_ref[...]
        new = sums_ref[...] / jnp.maximum(cnt, 1.0)
        c = c_ref[...]
        u_ref[...] = jnp.where(cnt > 0.0,
                               (c * p_ref[...] + new) / (c + 1.0),
                               p_ref[...])


def _dist_kernel(x_ref, u_ref, o_ref):
    x = x_ref[...]
    u = u_ref[...]
    ones_row = jnp.ones((1, EMBED_DIM), jnp.float32)
    d = jax.lax.dot_general(x, u, (((1,), (1,)), ((), ())),
                            precision=_PREC, preferred_element_type=jnp.float32)
    xsq = jax.lax.dot_general(x * x, ones_row, (((1,), (1,)), ((), ())),
                              precision=_PREC, preferred_element_type=jnp.float32)
    usq = jax.lax.dot_general(ones_row, u * u, (((1,), (1,)), ((), ())),
                              precision=_PREC, preferred_element_type=jnp.float32)
    o_ref[...] = -jnp.maximum(xsq + usq - 2.0 * d, 0.0)


def kernel(x, y_true, prototypes, counter):
    y2 = y_true.reshape(N_TOKENS, 1)
    c2 = counter.reshape(NUM_CLASSES, 1)

    u = pl.pallas_call(
        _update_kernel,
        grid=(A_STEPS,),
        in_specs=[
            pl.BlockSpec((A_BLK, EMBED_DIM), lambda i: (i, 0)),
            pl.BlockSpec((A_BLK, 1), lambda i: (i, 0)),
            pl.BlockSpec((NUM_CLASSES, EMBED_DIM), lambda i: (0, 0)),
            pl.BlockSpec((NUM_CLASSES, 1), lambda i: (0, 0)),
        ],
        out_specs=pl.BlockSpec((NUM_CLASSES, EMBED_DIM), lambda i: (0, 0)),
        out_shape=jax.ShapeDtypeStruct((NUM_CLASSES, EMBED_DIM), jnp.float32),
        scratch_shapes=[
            pltpu.VMEM((NUM_CLASSES, EMBED_DIM), jnp.float32),
            pltpu.VMEM((NUM_CLASSES, EMBED_DIM), jnp.float32),
        ],
    )(x, y2, prototypes, c2)

    out = pl.pallas_call(
        _dist_kernel,
        grid=(B_STEPS,),
        in_specs=[
            pl.BlockSpec((B_BLK, EMBED_DIM), lambda i: (i, 0)),
            pl.BlockSpec((NUM_CLASSES, EMBED_DIM), lambda i: (0, 0)),
        ],
        out_specs=pl.BlockSpec((B_BLK, NUM_CLASSES), lambda i: (i, 0)),
        out_shape=jax.ShapeDtypeStruct((N_TOKENS, NUM_CLASSES), jnp.float32),
    )(x, u)
    return out
